# baseline (speedup 1.0000x reference)
import functools

import jax
import jax.numpy as jnp
from jax import lax
from jax.experimental import pallas as pl
from jax.experimental.pallas import tpu as pltpu
from jax.experimental.pallas import tpu_sc as plsc

N_USERS = 25000
N_NODES = 50000
H = 32

NS = 16
NC = 2
NACC = 50008
RT = N_NODES // NS
WB = 125
NWB = RT // WB
CH = 256
NPC = 8
SUP = CH * NPC
EPT = 51200
NSUP = EPT // SUP
EP = EPT * NS
N_EDGES = 800000

_mesh = plsc.VectorSubcoreMesh(core_axis_name="c", subcore_axis_name="s")


@functools.partial(
    pl.kernel,
    out_type=[
        jax.ShapeDtypeStruct((NC * N_NODES, H), jnp.float32),
        jax.ShapeDtypeStruct((NC * N_NODES, H), jnp.bfloat16),
    ],
    mesh=_mesh,
    scratch_types=[
        pltpu.VMEM_SHARED((NACC, H), jnp.float32),
        pltpu.VMEM((SUP,), jnp.int32),
        pltpu.VMEM((NPC, CH), jnp.int32),
        pltpu.VMEM((SUP,), jnp.float32),
        pltpu.VMEM((2, CH, H), jnp.bfloat16),
        pltpu.VMEM((2, CH, H), jnp.float32),
        pltpu.SemaphoreType.DMA,
        pltpu.SemaphoreType.DMA,
        pltpu.SemaphoreType.DMA,
        pltpu.SemaphoreType.DMA,
    ],
    compiler_params=pltpu.CompilerParams(use_tc_tiling_on_sc=False, needs_layout_passes=False),
)
def _lightgcn_sc(emb_pk, emb2, col_hbm, row2d, w_hbm, out, ebuf, acc, colv,
                 rowv, wv, gbuf, sbuf, sem0, sem1, sem_i, sem_s):
    c = lax.axis_index("c")
    s = lax.axis_index("s")
    coff = c * N_NODES
    row_base = s * RT
    sems = (sem0, sem1)

    def layer(src_tab, old_tab, is_last):
        def zbody(r, _):
            z = jnp.zeros((16,), jnp.float32)
            sbuf[0, r, 0:16] = z
            sbuf[0, r, 16:32] = z
            return 0
        lax.fori_loop(0, WB, zbody, 0)
        for m in range(NWB):
            pltpu.sync_copy(sbuf.at[0].at[pl.ds(0, WB)],
                            acc.at[pl.ds(row_base + m * WB, WB)])

        @pl.when(s == 0)
        def _():
            pltpu.sync_copy(sbuf.at[0].at[pl.ds(0, 8)],
                            acc.at[pl.ds(N_NODES, 8)])
        plsc.subcore_barrier()

        def sup_body(t, _):
            erow = s * (EPT // CH) + t * NPC
            ebase = s * EPT + t * SUP
            idx_descs = [
                pltpu.async_copy(col_hbm.at[pl.ds(ebase, SUP)], colv, sem_i),
                pltpu.async_copy(row2d.at[pl.ds(erow, NPC)], rowv, sem_i),
                pltpu.async_copy(w_hbm.at[pl.ds(ebase, SUP)], wv, sem_i),
            ]
            for d in idx_descs:
                d.wait()
            def cadd(j, _):
                for i in range(4):
                    base = j * 64 + i * 16
                    colv[pl.ds(base, 16)] = colv[pl.ds(base, 16)] + coff
                return 0
            lax.fori_loop(0, SUP // 64, cadd, 0)

            def fire(cc):
                slot = cc % 2
                return [
                    pltpu.async_copy(
                        src_tab.at[colv.at[pl.ds(cc * CH, CH)]],
                        gbuf.at[slot], sems[slot])
                ]

            descs = fire(0)
            sc_prev = None
            for cc in range(NPC):
                slot = cc % 2
                if sc_prev is not None:
                    for d in sc_prev:
                        d.wait()
                nxt = fire(cc + 1) if cc + 1 < NPC else None
                for d in descs:
                    d.wait()
                descs = nxt

                def scale(g, _):
                    wvec = wv[pl.ds(cc * CH + g * 16, 16)]
                    for u in range(16):
                        r = g * 16 + u
                        ws = wvec.at[jnp.full((16,), u, jnp.int32)].get(
                            mode="promise_in_bounds")
                        a, b = plsc.unpack(
                            gbuf[slot, r, 0:32],
                            format=plsc.PackFormat.INTERLEAVED)
                        sbuf[slot, r, 0:16] = a * ws
                        sbuf[slot, r, 16:32] = b * ws
                    return 0
                lax.fori_loop(0, CH // 16, scale, 0)

                sc_prev = [
                    pltpu.async_copy(sbuf.at[slot],
                                     acc.at[rowv.at[cc]], sem_s,
                                     add=True)
                ]
            for d in sc_prev:
                d.wait()
            return 0

        lax.fori_loop(0, NSUP, sup_body, 0)
        plsc.subcore_barrier()

        for m in range(NWB):
            off = row_base + m * WB
            hoff = coff + off
            pltpu.sync_copy(acc.at[pl.ds(off, WB)],
                            sbuf.at[0].at[pl.ds(0, WB)])
            pltpu.sync_copy(old_tab.at[pl.ds(hoff, WB)],
                            sbuf.at[1].at[pl.ds(0, WB)])

            def accum(r, _):
                n0 = sbuf[0, r, 0:16]
                n1 = sbuf[0, r, 16:32]
                a0 = sbuf[1, r, 0:16] + n0
                a1 = sbuf[1, r, 16:32] + n1
                if is_last:
                    third = jnp.float32(1.0 / 3.0)
                    a0 = a0 * third
                    a1 = a1 * third
                else:
                    gbuf[0, r, 0:32] = plsc.pack(
                        n0, n1, format=plsc.PackFormat.INTERLEAVED)
                sbuf[1, r, 0:16] = a0
                sbuf[1, r, 16:32] = a1
                return 0
            lax.fori_loop(0, WB, accum, 0)

            pltpu.sync_copy(sbuf.at[1].at[pl.ds(0, WB)],
                            out.at[pl.ds(hoff, WB)])
            if not is_last:
                pltpu.sync_copy(gbuf.at[0].at[pl.ds(0, WB)],
                                ebuf.at[pl.ds(hoff, WB)])
        plsc.subcore_barrier()

    layer(emb_pk, emb2, is_last=False)
    layer(ebuf, out, is_last=True)


def kernel(embedding, edge_weight, edge_index):
    row = edge_index[0].astype(jnp.int32)
    col = edge_index[1].astype(jnp.int32)
    w = edge_weight.astype(jnp.float32)

    emb2 = jnp.stack([embedding[:, :H], embedding[:, H:]], axis=0)
    emb2 = emb2.reshape(NC * N_NODES, H)
    emb_pk = jnp.stack([emb2[:, 0:16], emb2[:, 16:32]], axis=2)
    emb_pk = emb_pk.reshape(NC * N_NODES, H).astype(jnp.bfloat16)

    colp = jnp.zeros((EP,), jnp.int32).at[:N_EDGES].set(col)
    rowp = jnp.full((EP,), N_NODES, jnp.int32).at[:N_EDGES].set(row).reshape(EP // CH, CH)
    wp = jnp.zeros((EP,), jnp.float32).at[:N_EDGES].set(w)

    out, _ = _lightgcn_sc(emb_pk, emb2, colp, rowp, wp)
    halves = out.reshape(NC, N_NODES, H)
    e_final = jnp.concatenate([halves[0], halves[1]], axis=1)
    return (e_final[:N_USERS], e_final[N_USERS:])


# speedup vs baseline: 1.0986x; 1.0986x over previous
import functools

import jax
import jax.numpy as jnp
from jax import lax
from jax.experimental import pallas as pl
from jax.experimental.pallas import tpu as pltpu
from jax.experimental.pallas import tpu_sc as plsc

N_USERS = 25000
N_NODES = 50000
H = 32
HP = 16

NS = 16
NC = 2
NACC = 50008
RT = N_NODES // NS
WB = 125
NWB = RT // WB
CH = 256
NPC = 8
SUP = CH * NPC
EPT = 51200
NSUP = EPT // SUP
EP = EPT * NS
N_EDGES = 800000

_mesh = plsc.VectorSubcoreMesh(core_axis_name="c", subcore_axis_name="s")


@functools.partial(
    pl.kernel,
    out_type=[
        jax.ShapeDtypeStruct((NC * N_NODES, H), jnp.float32),
        jax.ShapeDtypeStruct((NC * N_NODES, HP), jnp.int32),
    ],
    mesh=_mesh,
    scratch_types=[
        pltpu.VMEM_SHARED((NACC, H), jnp.float32),
        pltpu.VMEM((SUP,), jnp.int32),
        pltpu.VMEM((NPC, CH), jnp.int32),
        pltpu.VMEM((SUP,), jnp.float32),
        pltpu.VMEM((2, CH, HP), jnp.int32),
        pltpu.VMEM((2, CH, H), jnp.float32),
        pltpu.SemaphoreType.DMA,
        pltpu.SemaphoreType.DMA,
        pltpu.SemaphoreType.DMA,
        pltpu.SemaphoreType.DMA,
    ],
    compiler_params=pltpu.CompilerParams(use_tc_tiling_on_sc=False, needs_layout_passes=False),
)
def _lightgcn_sc(emb_pk, emb2, col_hbm, row2d, w_hbm, out, ebuf, acc, colv,
                 rowv, wv, gbuf, sbuf, sem0, sem1, sem_i, sem_s):
    c = lax.axis_index("c")
    s = lax.axis_index("s")
    coff = c * N_NODES
    row_base = s * RT
    sems = (sem0, sem1)

    def layer(src_tab, old_tab, is_last):
        def zbody(r, _):
            z = jnp.zeros((16,), jnp.float32)
            sbuf[0, r, 0:16] = z
            sbuf[0, r, 16:32] = z
            return 0
        lax.fori_loop(0, WB, zbody, 0)
        for m in range(NWB):
            pltpu.sync_copy(sbuf.at[0].at[pl.ds(0, WB)],
                            acc.at[pl.ds(row_base + m * WB, WB)])

        @pl.when(s == 0)
        def _():
            pltpu.sync_copy(sbuf.at[0].at[pl.ds(0, 8)],
                            acc.at[pl.ds(N_NODES, 8)])
        plsc.subcore_barrier()

        def sup_body(t, _):
            erow = s * (EPT // CH) + t * NPC
            ebase = s * EPT + t * SUP
            idx_descs = [
                pltpu.async_copy(col_hbm.at[pl.ds(ebase, SUP)], colv, sem_i),
                pltpu.async_copy(row2d.at[pl.ds(erow, NPC)], rowv, sem_i),
                pltpu.async_copy(w_hbm.at[pl.ds(ebase, SUP)], wv, sem_i),
            ]
            for d in idx_descs:
                d.wait()
            def cadd(j, _):
                for i in range(4):
                    base = j * 64 + i * 16
                    colv[pl.ds(base, 16)] = colv[pl.ds(base, 16)] + coff
                return 0
            lax.fori_loop(0, SUP // 64, cadd, 0)

            def fire(cc):
                slot = cc % 2
                return [
                    pltpu.async_copy(
                        src_tab.at[colv.at[pl.ds(cc * CH, CH)]],
                        gbuf.at[slot], sems[slot])
                ]

            descs = fire(0)
            sc_prev = None
            for cc in range(NPC):
                slot = cc % 2
                if sc_prev is not None:
                    for d in sc_prev:
                        d.wait()
                nxt = fire(cc + 1) if cc + 1 < NPC else None
                for d in descs:
                    d.wait()
                descs = nxt

                def scale(g, _):
                    wvec = wv[pl.ds(cc * CH + g * 16, 16)]
                    for u in range(16):
                        r = g * 16 + u
                        ws = wvec.at[jnp.full((16,), u, jnp.int32)].get(
                            mode="promise_in_bounds")
                        pk = gbuf[slot, r, 0:16]
                        a = plsc.bitcast(lax.shift_left(pk, 16), jnp.float32)
                        b = plsc.bitcast(
                            lax.bitwise_and(pk, jnp.int32(-65536)),
                            jnp.float32)
                        sbuf[slot, r, 0:16] = a * ws
                        sbuf[slot, r, 16:32] = b * ws
                    return 0
                lax.fori_loop(0, CH // 16, scale, 0)

                sc_prev = [
                    pltpu.async_copy(sbuf.at[slot],
                                     acc.at[rowv.at[cc]], sem_s,
                                     add=True)
                ]
            for d in sc_prev:
                d.wait()
            return 0

        lax.fori_loop(0, NSUP, sup_body, 0)
        plsc.subcore_barrier()

        for m in range(NWB):
            off = row_base + m * WB
            hoff = coff + off
            pltpu.sync_copy(acc.at[pl.ds(off, WB)],
                            sbuf.at[0].at[pl.ds(0, WB)])
            pltpu.sync_copy(old_tab.at[pl.ds(hoff, WB)],
                            sbuf.at[1].at[pl.ds(0, WB)])

            def accum(r, _):
                n0 = sbuf[0, r, 0:16]
                n1 = sbuf[0, r, 16:32]
                a0 = sbuf[1, r, 0:16] + n0
                a1 = sbuf[1, r, 16:32] + n1
                if is_last:
                    third = jnp.float32(1.0 / 3.0)
                    a0 = a0 * third
                    a1 = a1 * third
                else:
                    ua = lax.shift_right_logical(
                        plsc.bitcast(n0, jnp.int32) + jnp.int32(0x8000), 16)
                    ub = lax.bitwise_and(
                        plsc.bitcast(n1, jnp.int32) + jnp.int32(0x8000),
                        jnp.int32(-65536))
                    gbuf[0, r, 0:16] = lax.bitwise_or(ua, ub)
                sbuf[1, r, 0:16] = a0
                sbuf[1, r, 16:32] = a1
                return 0
            lax.fori_loop(0, WB, accum, 0)

            pltpu.sync_copy(sbuf.at[1].at[pl.ds(0, WB)],
                            out.at[pl.ds(hoff, WB)])
            if not is_last:
                pltpu.sync_copy(gbuf.at[0].at[pl.ds(0, WB)],
                                ebuf.at[pl.ds(hoff, WB)])
        plsc.subcore_barrier()

    layer(emb_pk, emb2, is_last=False)
    layer(ebuf, out, is_last=True)


def kernel(embedding, edge_weight, edge_index):
    row = edge_index[0].astype(jnp.int32)
    col = edge_index[1].astype(jnp.int32)
    w = edge_weight.astype(jnp.float32)

    emb2 = jnp.stack([embedding[:, :H], embedding[:, H:]], axis=0)
    emb2 = emb2.reshape(NC * N_NODES, H)
    ea = jax.lax.bitcast_convert_type(emb2[:, 0:16], jnp.int32)
    eb = jax.lax.bitcast_convert_type(emb2[:, 16:32], jnp.int32)
    emb_pk = jax.lax.bitwise_or(
        jax.lax.shift_right_logical(ea + 0x8000, 16),
        jax.lax.bitwise_and(eb + 0x8000, jnp.int32(-65536)))

    colp = jnp.zeros((EP,), jnp.int32).at[:N_EDGES].set(col)
    rowp = jnp.full((EP,), N_NODES, jnp.int32).at[:N_EDGES].set(row).reshape(EP // CH, CH)
    wp = jnp.zeros((EP,), jnp.float32).at[:N_EDGES].set(w)

    out, _ = _lightgcn_sc(emb_pk, emb2, colp, rowp, wp)
    halves = out.reshape(NC, N_NODES, H)
    e_final = jnp.concatenate([halves[0], halves[1]], axis=1)
    return (e_final[:N_USERS], e_final[N_USERS:])


# speedup vs baseline: 1.9846x; 1.8065x over previous
import functools

import jax
import jax.numpy as jnp
from jax import lax
from jax.experimental import pallas as pl
from jax.experimental.pallas import tpu as pltpu
from jax.experimental.pallas import tpu_sc as plsc

N_USERS = 25000
N_NODES = 50000
D = 64
H = 32

NS = 16
NC = 2
NACC = 50008
RT = N_NODES // NS
WB = 125
NWB = RT // WB
CH = 256
NPC = 8
SUP = CH * NPC
EPT = 800000 // NS
NSUP = EPT // SUP
TAIL = EPT - NSUP * SUP
TNPC = (TAIL + CH - 1) // CH
TFULL = TAIL // CH
TREM = TAIL - TFULL * CH

_mesh = plsc.VectorSubcoreMesh(core_axis_name="c", subcore_axis_name="s")


@functools.partial(
    pl.kernel,
    out_type=[
        jax.ShapeDtypeStruct((N_NODES, D), jnp.float32),
        jax.ShapeDtypeStruct((NC * N_NODES, H), jnp.float32),
        jax.ShapeDtypeStruct((NC * N_NODES, H), jnp.float32),
    ],
    mesh=_mesh,
    scratch_types=[
        pltpu.VMEM_SHARED((NACC, H), jnp.float32),
        pltpu.VMEM((SUP,), jnp.int32),
        pltpu.VMEM((NPC, CH), jnp.int32),
        pltpu.VMEM((SUP,), jnp.float32),
        pltpu.VMEM((2, CH, H), jnp.float32),
        pltpu.VMEM((WB, H), jnp.float32),
        pltpu.SemaphoreType.DMA,
        pltpu.SemaphoreType.DMA,
        pltpu.SemaphoreType.DMA,
        pltpu.SemaphoreType.DMA,
    ],
    compiler_params=pltpu.CompilerParams(use_tc_tiling_on_sc=False),
)
def _lightgcn_sc(emb, eidx, w_hbm, out, ftab, ebuf, acc, colv, rowv, wv,
                 gbuf, abuf, sem0, sem1, sem_i, sem_s):
    c = lax.axis_index("c")
    s = lax.axis_index("s")
    coff = c * N_NODES
    row_base = s * RT
    sems = (sem0, sem1)

    for m in range(NWB):
        r0 = row_base + m * WB
        pltpu.sync_copy(emb.at[pl.ds(r0, WB), pl.ds(c * H, H)],
                        gbuf.at[0].at[pl.ds(0, WB)])
        pltpu.sync_copy(gbuf.at[0].at[pl.ds(0, WB)],
                        ftab.at[pl.ds(coff + r0, WB)])
    plsc.subcore_barrier()

    def fill16(ref, base, n16, val, dtype):
        def body(i, _):
            ref[pl.ds(base + i * 16, 16)] = jnp.full((16,), val, dtype)
            return 0
        lax.fori_loop(0, n16, body, 0)

    def layer(src_tab, old_tab, is_last):
        def zbody(r, _):
            z = jnp.zeros((16,), jnp.float32)
            abuf[r, 0:16] = z
            abuf[r, 16:32] = z
            return 0
        lax.fori_loop(0, WB, zbody, 0)
        for m in range(NWB):
            pltpu.sync_copy(abuf, acc.at[pl.ds(row_base + m * WB, WB)])

        @pl.when(s == 0)
        def _():
            pltpu.sync_copy(abuf.at[pl.ds(0, 8)], acc.at[pl.ds(N_NODES, 8)])
        plsc.subcore_barrier()

        def run_chunks(npc):
            def fire(cc):
                slot = cc % 2
                return pltpu.async_copy(
                    src_tab.at[colv.at[pl.ds(cc * CH, CH)]],
                    gbuf.at[slot], sems[slot])

            desc = fire(0)
            sc_prev = None
            for cc in range(npc):
                slot = cc % 2
                if sc_prev is not None:
                    sc_prev.wait()
                nxt = fire(cc + 1) if cc + 1 < npc else None
                desc.wait()
                desc = nxt

                def scale(g, _):
                    wvec = wv[pl.ds(cc * CH + g * 16, 16)]
                    for u in range(16):
                        r = g * 16 + u
                        ws = wvec.at[jnp.full((16,), u, jnp.int32)].get(
                            mode="promise_in_bounds")
                        gbuf[slot, r, 0:16] = gbuf[slot, r, 0:16] * ws
                        gbuf[slot, r, 16:32] = gbuf[slot, r, 16:32] * ws
                    return 0
                lax.fori_loop(0, CH // 16, scale, 0)

                sc_prev = pltpu.async_copy(gbuf.at[slot],
                                           acc.at[rowv.at[cc]], sem_s,
                                           add=True)
            sc_prev.wait()

        def stage_idx(ebase, n):
            descs = [
                pltpu.async_copy(eidx.at[1, pl.ds(ebase, n)],
                                 colv.at[pl.ds(0, n)], sem_i),
                pltpu.async_copy(w_hbm.at[pl.ds(ebase, n)],
                                 wv.at[pl.ds(0, n)], sem_i),
            ]
            nrow = (n + CH - 1) // CH
            for j in range(nrow):
                sz = min(CH, n - j * CH)
                descs.append(
                    pltpu.async_copy(eidx.at[0, pl.ds(ebase + j * CH, sz)],
                                     rowv.at[j].at[pl.ds(0, sz)], sem_i))
            for d in descs:
                d.wait()
            def cadd(j, _):
                colv[pl.ds(j * 16, 16)] = colv[pl.ds(j * 16, 16)] + coff
                return 0
            lax.fori_loop(0, (n + 15) // 16, cadd, 0)

        def sup_body(t, _):
            ebase = s * EPT + t * SUP
            stage_idx(ebase, SUP)
            run_chunks(NPC)
            return 0
        lax.fori_loop(0, NSUP, sup_body, 0)

        stage_idx(s * EPT + NSUP * SUP, TAIL)
        fill16(colv, TAIL, (TNPC * CH - TAIL) // 16, coff, jnp.int32)
        fill16(wv, TAIL, (TNPC * CH - TAIL) // 16, 0.0, jnp.float32)

        def rfill(i, _):
            rowv[TFULL, pl.ds(TREM + i * 16, 16)] = jnp.full(
                (16,), N_NODES, jnp.int32)
            return 0
        lax.fori_loop(0, (CH - TREM) // 16, rfill, 0)
        run_chunks(TNPC)

        plsc.subcore_barrier()

        for m in range(NWB):
            off = row_base + m * WB
            hoff = coff + off
            pltpu.sync_copy(acc.at[pl.ds(off, WB)],
                            gbuf.at[0].at[pl.ds(0, WB)])
            pltpu.sync_copy(old_tab.at[pl.ds(hoff, WB)], abuf)

            def accum(r, _):
                a0 = abuf[r, 0:16] + gbuf[0, r, 0:16]
                a1 = abuf[r, 16:32] + gbuf[0, r, 16:32]
                if is_last:
                    third = jnp.float32(1.0 / 3.0)
                    a0 = a0 * third
                    a1 = a1 * third
                abuf[r, 0:16] = a0
                abuf[r, 16:32] = a1
                return 0
            lax.fori_loop(0, WB, accum, 0)

            if is_last:
                pltpu.sync_copy(abuf,
                                out.at[pl.ds(off, WB), pl.ds(c * H, H)])
            else:
                pltpu.sync_copy(abuf, ftab.at[pl.ds(hoff, WB)])
                pltpu.sync_copy(gbuf.at[0].at[pl.ds(0, WB)],
                                ebuf.at[pl.ds(hoff, WB)])
        plsc.subcore_barrier()

    layer(ftab, ftab, is_last=False)
    layer(ebuf, ftab, is_last=True)


def kernel(embedding, edge_weight, edge_index):
    eidx = edge_index.astype(jnp.int32)
    w = edge_weight.astype(jnp.float32)
    out, _, _ = _lightgcn_sc(embedding.astype(jnp.float32), eidx, w)
    return (out[:N_USERS], out[N_USERS:])


# speedup vs baseline: 2.0522x; 1.0341x over previous
import functools

import jax
import jax.numpy as jnp
from jax import lax
from jax.experimental import pallas as pl
from jax.experimental.pallas import tpu as pltpu
from jax.experimental.pallas import tpu_sc as plsc

N_USERS = 25000
N_NODES = 50000
D = 64
H = 32

NS = 16
NC = 2
NACC = 50008
RT = N_NODES // NS
WB = 125
NWB = RT // WB
CH = 256
NPC = 8
SUP = CH * NPC
EPT = 800000 // NS
NSUP = EPT // SUP
TAIL = EPT - NSUP * SUP
TNPC = (TAIL + CH - 1) // CH
TFULL = TAIL // CH
TREM = TAIL - TFULL * CH

_mesh = plsc.VectorSubcoreMesh(core_axis_name="c", subcore_axis_name="s")


@functools.partial(
    pl.kernel,
    out_type=[
        jax.ShapeDtypeStruct((N_USERS, D), jnp.float32),
        jax.ShapeDtypeStruct((N_NODES - N_USERS, D), jnp.float32),
        jax.ShapeDtypeStruct((NC * N_NODES, H), jnp.float32),
        jax.ShapeDtypeStruct((NC * N_NODES, H), jnp.float32),
    ],
    mesh=_mesh,
    scratch_types=[
        pltpu.VMEM_SHARED((NACC, H), jnp.float32),
        pltpu.VMEM((SUP,), jnp.int32),
        pltpu.VMEM((NPC, CH), jnp.int32),
        pltpu.VMEM((SUP,), jnp.float32),
        pltpu.VMEM((2, CH, H), jnp.float32),
        pltpu.VMEM((WB, H), jnp.float32),
        pltpu.SemaphoreType.DMA,
        pltpu.SemaphoreType.DMA,
        pltpu.SemaphoreType.DMA,
        pltpu.SemaphoreType.DMA,
    ],
    compiler_params=pltpu.CompilerParams(use_tc_tiling_on_sc=False),
)
def _lightgcn_sc(emb, eidx, w_hbm, out_u, out_i, ftab, ebuf, acc, colv, rowv, wv,
                 gbuf, abuf, sem0, sem1, sem_i, sem_s):
    c = lax.axis_index("c")
    s = lax.axis_index("s")
    coff = c * N_NODES
    row_base = s * RT
    sems = (sem0, sem1)

    for m in range(NWB):
        r0 = row_base + m * WB
        pltpu.sync_copy(emb.at[pl.ds(r0, WB), pl.ds(c * H, H)],
                        gbuf.at[0].at[pl.ds(0, WB)])
        pltpu.sync_copy(gbuf.at[0].at[pl.ds(0, WB)],
                        ftab.at[pl.ds(coff + r0, WB)])
    plsc.subcore_barrier()

    def fill16(ref, base, n16, val, dtype):
        def body(i, _):
            ref[pl.ds(base + i * 16, 16)] = jnp.full((16,), val, dtype)
            return 0
        lax.fori_loop(0, n16, body, 0)

    def layer(src_tab, old_tab, is_last):
        def zbody(r, _):
            z = jnp.zeros((16,), jnp.float32)
            abuf[r, 0:16] = z
            abuf[r, 16:32] = z
            return 0
        lax.fori_loop(0, WB, zbody, 0)
        for m in range(NWB):
            pltpu.sync_copy(abuf, acc.at[pl.ds(row_base + m * WB, WB)])

        @pl.when(s == 0)
        def _():
            pltpu.sync_copy(abuf.at[pl.ds(0, 8)], acc.at[pl.ds(N_NODES, 8)])
        plsc.subcore_barrier()

        def run_chunks(npc):
            def fire(cc):
                slot = cc % 2
                return pltpu.async_copy(
                    src_tab.at[colv.at[pl.ds(cc * CH, CH)]],
                    gbuf.at[slot], sems[slot])

            desc = fire(0)
            sc_prev = None
            for cc in range(npc):
                slot = cc % 2
                if sc_prev is not None:
                    sc_prev.wait()
                nxt = fire(cc + 1) if cc + 1 < npc else None
                desc.wait()
                desc = nxt

                def scale(g, _):
                    wvec = wv[pl.ds(cc * CH + g * 16, 16)]
                    for u in range(16):
                        r = g * 16 + u
                        ws = wvec.at[jnp.full((16,), u, jnp.int32)].get(
                            mode="promise_in_bounds")
                        gbuf[slot, r, 0:16] = gbuf[slot, r, 0:16] * ws
                        gbuf[slot, r, 16:32] = gbuf[slot, r, 16:32] * ws
                    return 0
                lax.fori_loop(0, CH // 16, scale, 0)

                sc_prev = pltpu.async_copy(gbuf.at[slot],
                                           acc.at[rowv.at[cc]], sem_s,
                                           add=True)
            sc_prev.wait()

        def stage_idx(ebase, n):
            descs = [
                pltpu.async_copy(eidx.at[1, pl.ds(ebase, n)],
                                 colv.at[pl.ds(0, n)], sem_i),
                pltpu.async_copy(w_hbm.at[pl.ds(ebase, n)],
                                 wv.at[pl.ds(0, n)], sem_i),
            ]
            nrow = (n + CH - 1) // CH
            for j in range(nrow):
                sz = min(CH, n - j * CH)
                descs.append(
                    pltpu.async_copy(eidx.at[0, pl.ds(ebase + j * CH, sz)],
                                     rowv.at[j].at[pl.ds(0, sz)], sem_i))
            for d in descs:
                d.wait()
            def cadd(j, _):
                colv[pl.ds(j * 16, 16)] = colv[pl.ds(j * 16, 16)] + coff
                return 0
            lax.fori_loop(0, (n + 15) // 16, cadd, 0)

        def sup_body(t, _):
            ebase = s * EPT + t * SUP
            stage_idx(ebase, SUP)
            run_chunks(NPC)
            return 0
        lax.fori_loop(0, NSUP, sup_body, 0)

        stage_idx(s * EPT + NSUP * SUP, TAIL)
        fill16(colv, TAIL, (TNPC * CH - TAIL) // 16, coff, jnp.int32)
        fill16(wv, TAIL, (TNPC * CH - TAIL) // 16, 0.0, jnp.float32)

        def rfill(i, _):
            rowv[TFULL, pl.ds(TREM + i * 16, 16)] = jnp.full(
                (16,), N_NODES, jnp.int32)
            return 0
        lax.fori_loop(0, (CH - TREM) // 16, rfill, 0)
        run_chunks(TNPC)

        plsc.subcore_barrier()

        for m in range(NWB):
            off = row_base + m * WB
            hoff = coff + off
            pltpu.sync_copy(acc.at[pl.ds(off, WB)],
                            gbuf.at[0].at[pl.ds(0, WB)])
            pltpu.sync_copy(old_tab.at[pl.ds(hoff, WB)], abuf)

            def accum(r, _):
                a0 = abuf[r, 0:16] + gbuf[0, r, 0:16]
                a1 = abuf[r, 16:32] + gbuf[0, r, 16:32]
                if is_last:
                    third = jnp.float32(1.0 / 3.0)
                    a0 = a0 * third
                    a1 = a1 * third
                abuf[r, 0:16] = a0
                abuf[r, 16:32] = a1
                return 0
            lax.fori_loop(0, WB, accum, 0)

            if is_last:
                @pl.when(s < NS // 2)
                def _():
                    pltpu.sync_copy(
                        abuf, out_u.at[pl.ds(off, WB), pl.ds(c * H, H)])

                @pl.when(s >= NS // 2)
                def _():
                    pltpu.sync_copy(
                        abuf,
                        out_i.at[pl.ds(off - N_USERS, WB), pl.ds(c * H, H)])
            else:
                pltpu.sync_copy(abuf, ftab.at[pl.ds(hoff, WB)])
                pltpu.sync_copy(gbuf.at[0].at[pl.ds(0, WB)],
                                ebuf.at[pl.ds(hoff, WB)])
        plsc.subcore_barrier()

    layer(ftab, ftab, is_last=False)
    layer(ebuf, ftab, is_last=True)


def kernel(embedding, edge_weight, edge_index):
    eidx = edge_index.astype(jnp.int32)
    w = edge_weight.astype(jnp.float32)
    out_u, out_i, _, _ = _lightgcn_sc(embedding.astype(jnp.float32), eidx, w)
    return (out_u, out_i)
